# baseline (speedup 1.0000x reference)
import functools

import jax
import jax.numpy as jnp
from jax import lax
from jax.experimental import pallas as pl
from jax.experimental.pallas import tpu as pltpu
from jax.experimental.pallas import tpu_sc as plsc


_G = 64


_CA = 40


def _assign_body(x_ref, cb_ref, idx_ref, xout_ref, cba_ref, xba_ref):
    g, c, w = x_ref.shape
    k = cb_ref.shape[0]
    blk = g * w

    @pl.when(pl.program_id(0) == 0)
    def _prep():
        cb = cb_ref[...]
        c_sq = jnp.sum(cb * cb, axis=1, keepdims=True)
        hi = c_sq.astype(jnp.bfloat16).astype(jnp.float32)
        mid = (c_sq - hi).astype(jnp.bfloat16).astype(jnp.float32)
        lo = (c_sq - hi - mid).astype(jnp.bfloat16).astype(jnp.float32)
        cba_ref[:, 0:c] = -2.0 * cb
        cba_ref[:, c:c + 1] = hi
        cba_ref[:, c + 1:c + 2] = mid
        cba_ref[:, c + 2:c + 3] = lo
        cba_ref[:, c + 3:] = jnp.zeros((k, _CA - c - 3), jnp.float32)
        xba_ref[c:c + 3, :] = jnp.ones((3, blk), jnp.float32)
        xba_ref[c + 3:, :] = jnp.zeros((_CA - c - 3, blk), jnp.float32)

    xb3 = x_ref[...]
    xout_ref[...] = xb3
    xba_ref[0:c, :] = xb3.transpose(1, 0, 2).reshape(c, blk)
    scores = lax.dot_general(
        cba_ref[...], xba_ref[...], (((1,), (0,)), ((), ())),
        preferred_element_type=jnp.float32)
    idx = jnp.argmin(scores, axis=0).astype(jnp.int32)
    idx_ref[0, 0, :] = idx


def _assign(x3, codebook):
    gh, c, w = x3.shape
    n = gh * w
    k = codebook.shape[0]
    nb = gh // _G
    blk = _G * w
    idx3, xout3 = pl.pallas_call(
        _assign_body,
        grid=(nb,),
        in_specs=[
            pl.BlockSpec((_G, c, w), lambda i: (i, 0, 0)),
            pl.BlockSpec((k, c), lambda i: (0, 0)),
        ],
        out_specs=[
            pl.BlockSpec((1, 1, blk), lambda i: (i, 0, 0)),
            pl.BlockSpec((_G, c, w), lambda i: (i, 0, 0)),
        ],
        out_shape=[
            jax.ShapeDtypeStruct((nb, 1, blk), jnp.int32),
            jax.ShapeDtypeStruct((gh, c, w), jnp.float32),
        ],
        scratch_shapes=[
            pltpu.VMEM((k, _CA), jnp.float32),
            pltpu.VMEM((_CA, blk), jnp.float32),
        ],
    )(x3, codebook)
    return idx3.reshape(n), xout3


@functools.lru_cache(maxsize=None)
def _make_gather(k_, c_, gh, w_):
    info = plsc.get_sparse_core_info()
    nc, ns, nl = info.num_cores, info.num_subcores, info.num_lanes
    nw = nc * ns
    assert gh % nw == 0 and w_ % nl == 0
    rows_per_tile = gh // nw
    toks_per_tile = rows_per_tile * w_
    groups_per_row = w_ // nl
    mesh = plsc.VectorSubcoreMesh(core_axis_name="c", subcore_axis_name="s")

    @functools.partial(
        pl.kernel,
        mesh=mesh,
        out_type=jax.ShapeDtypeStruct((gh * c_ * w_,), jnp.float32),
        scratch_types=[
            pltpu.VMEM((k_ * c_,), jnp.float32),
            pltpu.VMEM((toks_per_tile,), jnp.int32),
            pltpu.VMEM((toks_per_tile * c_,), jnp.float32),
        ],
        compiler_params=pltpu.CompilerParams(
            use_tc_tiling_on_sc=False, needs_layout_passes=False),
    )
    def gather(table_hbm, idx_hbm, out_hbm, cb_v, idx_v, out_v):
        wid = lax.axis_index("s") * nc + lax.axis_index("c")
        base_tok = wid * toks_per_tile
        pltpu.sync_copy(table_hbm, cb_v)
        pltpu.sync_copy(idx_hbm.at[pl.ds(base_tok, toks_per_tile)], idx_v)

        def row_body(bh, carry):
            def grp_body(wg, carry2):
                tok0 = bh * w_ + wg * nl
                iv = idx_v[pl.ds(tok0, nl)]
                addr = iv * c_
                obase = bh * (c_ * w_) + wg * nl
                for cc in range(c_):
                    vals = plsc.load_gather(cb_v, [addr + cc])
                    out_v[pl.ds(obase + cc * w_, nl)] = vals
                return carry2
            return lax.fori_loop(0, groups_per_row, grp_body, carry)

        lax.fori_loop(0, rows_per_tile, row_body, 0)
        pltpu.sync_copy(
            out_v, out_hbm.at[pl.ds(base_tok * c_, toks_per_tile * c_)])

    return gather


def kernel(x, codebook):
    b, h, w, c = x.shape
    n = b * h * w
    x3 = jnp.transpose(x, (0, 1, 3, 2)).reshape(b * h, c, w)
    idx, xout3 = _assign(x3, codebook)
    y_flat = _make_gather(codebook.shape[0], c, b * h, w)(
        codebook.reshape(-1), idx)
    y = jnp.transpose(y_flat.reshape(b, h, c, w), (0, 1, 3, 2))
    xout = jnp.transpose(xout3.reshape(b, h, c, w), (0, 1, 3, 2))
    return (xout, y)


# speedup vs baseline: 1.2723x; 1.2723x over previous
import functools

import jax
import jax.numpy as jnp
from jax import lax
from jax.experimental import pallas as pl
from jax.experimental.pallas import tpu as pltpu
from jax.experimental.pallas import tpu_sc as plsc


_G = 64


_CA = 40


def _assign_body(x_ref, cb_ref, idx_ref, xout_ref, cba_ref, xba_ref):
    g, c, w = x_ref.shape
    k = cb_ref.shape[0]
    blk = g * w

    @pl.when(pl.program_id(0) == 0)
    def _prep():
        cb = cb_ref[...]
        c_sq = jnp.sum(cb * cb, axis=1, keepdims=True)
        hi = c_sq.astype(jnp.bfloat16).astype(jnp.float32)
        mid = (c_sq - hi).astype(jnp.bfloat16).astype(jnp.float32)
        lo = (c_sq - hi - mid).astype(jnp.bfloat16).astype(jnp.float32)
        cba_ref[:, 0:c] = -2.0 * cb
        cba_ref[:, c:c + 1] = hi
        cba_ref[:, c + 1:c + 2] = mid
        cba_ref[:, c + 2:c + 3] = lo
        cba_ref[:, c + 3:] = jnp.zeros((k, _CA - c - 3), jnp.float32)
        xba_ref[c:c + 3, :] = jnp.ones((3, blk), jnp.float32)
        xba_ref[c + 3:, :] = jnp.zeros((_CA - c - 3, blk), jnp.float32)

    xb3 = x_ref[...]
    xout_ref[...] = xb3
    xba_ref[0:c, :] = xb3.transpose(1, 0, 2).reshape(c, blk)
    scores = lax.dot_general(
        cba_ref[...], xba_ref[...], (((1,), (0,)), ((), ())),
        preferred_element_type=jnp.float32)
    idx = jnp.argmin(scores, axis=0).astype(jnp.int32)
    idx_ref[0, 0, :] = idx


def _assign(x3, codebook):
    gh, c, w = x3.shape
    n = gh * w
    k = codebook.shape[0]
    nb = gh // _G
    blk = _G * w
    idx3, xout3 = pl.pallas_call(
        _assign_body,
        grid=(nb,),
        in_specs=[
            pl.BlockSpec((_G, c, w), lambda i: (i, 0, 0)),
            pl.BlockSpec((k, c), lambda i: (0, 0)),
        ],
        out_specs=[
            pl.BlockSpec((1, 1, blk), lambda i: (i, 0, 0)),
            pl.BlockSpec((_G, c, w), lambda i: (i, 0, 0)),
        ],
        out_shape=[
            jax.ShapeDtypeStruct((nb, 1, blk), jnp.int32),
            jax.ShapeDtypeStruct((gh, c, w), jnp.float32),
        ],
        scratch_shapes=[
            pltpu.VMEM((k, _CA), jnp.float32),
            pltpu.VMEM((_CA, blk), jnp.float32),
        ],
    )(x3, codebook)
    return idx3.reshape(n), xout3


@functools.lru_cache(maxsize=None)
def _make_gather(k_, c_, gh, w_):
    info = plsc.get_sparse_core_info()
    nc, ns, nl = info.num_cores, info.num_subcores, info.num_lanes
    nw = nc * ns
    assert gh % nw == 0 and w_ % nl == 0
    rows_per_tile = gh // nw
    toks_per_tile = rows_per_tile * w_
    groups_per_row = w_ // nl
    mesh = plsc.VectorSubcoreMesh(core_axis_name="c", subcore_axis_name="s")

    @functools.partial(
        pl.kernel,
        mesh=mesh,
        out_type=jax.ShapeDtypeStruct((gh * c_ * w_,), jnp.float32),
        scratch_types=[
            pltpu.VMEM((k_ * c_,), jnp.float32),
            pltpu.VMEM((toks_per_tile,), jnp.int32),
            pltpu.VMEM((toks_per_tile * c_,), jnp.float32),
        ],
        compiler_params=pltpu.CompilerParams(
            use_tc_tiling_on_sc=False, needs_layout_passes=False),
    )
    def gather(table_hbm, idx_hbm, out_hbm, cb_v, idx_v, out_v):
        wid = lax.axis_index("s") * nc + lax.axis_index("c")
        base_tok = wid * toks_per_tile
        pltpu.sync_copy(table_hbm, cb_v)
        pltpu.sync_copy(idx_hbm.at[pl.ds(base_tok, toks_per_tile)], idx_v)

        n_groups = rows_per_tile * groups_per_row

        @plsc.parallel_loop(0, n_groups, unroll=2)
        def _grp(g):
            bh = g // groups_per_row
            wg = g % groups_per_row
            tok0 = pl.multiple_of(bh * w_ + wg * nl, nl)
            obase = pl.multiple_of(bh * (c_ * w_) + wg * nl, nl)
            iv = idx_v[pl.ds(tok0, nl)]
            addr = iv * c_
            for cc in range(c_):
                vals = plsc.load_gather(cb_v, [addr + cc])
                out_v[pl.ds(obase + cc * w_, nl)] = vals
        pltpu.sync_copy(
            out_v, out_hbm.at[pl.ds(base_tok * c_, toks_per_tile * c_)])

    return gather


def kernel(x, codebook):
    b, h, w, c = x.shape
    n = b * h * w
    x3 = jnp.transpose(x, (0, 1, 3, 2)).reshape(b * h, c, w)
    idx, xout3 = _assign(x3, codebook)
    y_flat = _make_gather(codebook.shape[0], c, b * h, w)(
        codebook.reshape(-1), idx)
    y = jnp.transpose(y_flat.reshape(b, h, c, w), (0, 1, 3, 2))
    xout = jnp.transpose(xout3.reshape(b, h, c, w), (0, 1, 3, 2))
    return (xout, y)
